# baseline (speedup 1.0000x reference)
import functools

import jax
import jax.numpy as jnp
from jax import lax
from jax.experimental import pallas as pl
from jax.experimental.pallas import tpu as pltpu
from jax.experimental.pallas import tpu_sc as plsc

D = 128
CIN = 32
COUT = 64
WLEN = 16
NOUT = 3 * COUT
XW = WLEN * CIN


def _conv_slab(x, w, bb, row):
    acc = None
    for p in range(WLEN - 3):
        h = jnp.dot(w, x[CIN * p:CIN * p + 4 * CIN, :],
                    preferred_element_type=jnp.float32)
        acc = h if acc is None else jnp.maximum(acc, h)
    h = jnp.dot(w[:, :3 * CIN], x[CIN * (WLEN - 3):XW, :],
                preferred_element_type=jnp.float32)
    acc = jnp.where(row < 2 * COUT, jnp.maximum(acc, h), acc)
    h = jnp.dot(w[:, :2 * CIN], x[CIN * (WLEN - 2):XW, :],
                preferred_element_type=jnp.float32)
    acc = jnp.where(row < COUT, jnp.maximum(acc, h), acc)
    return jnp.tanh(acc + bb)


def _conv_body(x_ref, w_ref, b_ref, char_ref, fwo_ref):
    nl, _, nb = x_ref.shape
    w = w_ref[...]
    bb = b_ref[...]
    row = lax.broadcasted_iota(jnp.int32, (NOUT, nb), 0)
    for k in range(nl):
        acc = _conv_slab(x_ref[k], w, bb, row)
        char_ref[k] = acc
        fwo_ref[k] = acc


def _conv_call(x_t, wbig_t, bcol, nl=1):
    l, _, nb = x_t.shape
    return pl.pallas_call(
        _conv_body,
        grid=(l // nl,),
        in_specs=[
            pl.BlockSpec((nl, XW, nb), lambda i: (i, 0, 0)),
            pl.BlockSpec((NOUT, 4 * CIN), lambda i: (0, 0)),
            pl.BlockSpec((NOUT, 1), lambda i: (0, 0)),
        ],
        out_specs=[
            pl.BlockSpec((nl, NOUT, nb), lambda i: (i, 0, 0)),
            pl.BlockSpec((nl, NOUT, nb), lambda i: (i, 0, 0)),
        ],
        out_shape=[
            jax.ShapeDtypeStruct((l, NOUT, nb), jnp.float32),
            jax.ShapeDtypeStruct((l, NOUT + D, nb), jnp.float32),
        ],
    )(x_t, wbig_t, bcol)


def _embed_fill_body(emb_ref, fwo_in_ref, fwo_ref):
    del fwo_in_ref
    nl = emb_ref.shape[0]
    j = pl.program_id(1)
    for k in range(nl):
        t = jnp.transpose(emb_ref[k])
        fwo_ref[k] = jnp.where(j == 0, t[:COUT], t[COUT:])


def _embed_fill_call(emb3, fwo_partial, nl=10):
    l, nb, _ = emb3.shape
    return pl.pallas_call(
        _embed_fill_body,
        grid=(l // nl, 2),
        in_specs=[
            pl.BlockSpec((nl, nb, D), lambda i, j: (i, 0, 0)),
            pl.BlockSpec((nl, 8, nb), lambda i, j: (i, 0, 0)),
        ],
        out_specs=pl.BlockSpec(
            (nl, COUT, nb), lambda i, j: (i, NOUT // COUT + j, 0)
        ),
        out_shape=jax.ShapeDtypeStruct((l, NOUT + D, nb), jnp.float32),
        input_output_aliases={1: 0},
    )(emb3, fwo_partial)


@functools.cache
def _make_gather(vocab, d, t):
    info = plsc.get_sparse_core_info()
    nw = info.num_cores * info.num_subcores
    t_per_w = t // nw
    ch = 400
    n_ch = t_per_w // ch
    mesh = plsc.VectorSubcoreMesh(core_axis_name="c", subcore_axis_name="s")

    @functools.partial(
        pl.kernel,
        mesh=mesh,
        out_type=jax.ShapeDtypeStruct((t, d), jnp.float32),
        scratch_types=[
            pltpu.VMEM((t_per_w,), jnp.int32),
            pltpu.VMEM((2, ch, d), jnp.float32),
            pltpu.SemaphoreType.DMA,
            pltpu.SemaphoreType.DMA,
        ],
    )
    def gather_k(idx_hbm, table_hbm, out_hbm, idx_v, rows_v, sem0, sem1):
        wid = lax.axis_index("s") * info.num_cores + lax.axis_index("c")
        base = wid * t_per_w
        pltpu.sync_copy(idx_hbm.at[pl.ds(base, t_per_w)], idx_v)
        sems = (sem0, sem1)
        cps = [None, None]
        cps[0] = pltpu.async_copy(
            table_hbm.at[idx_v.at[pl.ds(0, ch)]], rows_v.at[0], sems[0]
        )
        for c in range(n_ch):
            nxt = c + 1
            if nxt < n_ch:
                cps[nxt % 2] = pltpu.async_copy(
                    table_hbm.at[idx_v.at[pl.ds(nxt * ch, ch)]],
                    rows_v.at[nxt % 2],
                    sems[nxt % 2],
                )
            cps[c % 2].wait()
            pltpu.sync_copy(rows_v.at[c % 2], out_hbm.at[pl.ds(base + c * ch, ch)])

    return gather_k


def kernel(x0_word_ids, x1_char_feats, table, W2, b2, W3, b3, W4, b4):
    b, l = x0_word_ids.shape
    t = b * l
    idx = jnp.transpose(x0_word_ids).reshape(t).astype(jnp.int32)
    emb_flat = _make_gather(table.shape[0], D, t)(idx, table)
    emb3 = emb_flat.reshape(l, b, D)

    w2p = jnp.pad(W2, ((0, 2 * CIN), (0, 0)))
    w3p = jnp.pad(W3, ((0, CIN), (0, 0)))
    wbig_t = jnp.transpose(jnp.concatenate([w2p, w3p, W4], axis=1))
    bcol = jnp.concatenate([b2, b3, b4]).reshape(NOUT, 1)

    x_t = jnp.transpose(x1_char_feats, (1, 2, 3, 0)).reshape(l, XW, b)
    char_t, fwo_partial = _conv_call(x_t, wbig_t, bcol, nl=5)
    fwo_t = _embed_fill_call(emb3, fwo_partial)

    return (
        jnp.transpose(emb3, (1, 0, 2)),
        jnp.transpose(char_t, (2, 0, 1)),
        jnp.transpose(fwo_t, (2, 0, 1)),
    )


# speedup vs baseline: 1.1093x; 1.1093x over previous
import functools

import jax
import jax.numpy as jnp
from jax import lax
from jax.experimental import pallas as pl
from jax.experimental.pallas import tpu as pltpu
from jax.experimental.pallas import tpu_sc as plsc

D = 128
CIN = 32
COUT = 64
WLEN = 16
NOUT = 3 * COUT
XW = WLEN * CIN


def _conv_slab(x, w, bb, row):
    acc = None
    for p in range(WLEN - 3):
        h = jnp.dot(w, x[CIN * p:CIN * p + 4 * CIN, :],
                    preferred_element_type=jnp.float32)
        acc = h if acc is None else jnp.maximum(acc, h)
    h = jnp.dot(w[:, :3 * CIN], x[CIN * (WLEN - 3):XW, :],
                preferred_element_type=jnp.float32)
    acc = jnp.where(row < 2 * COUT, jnp.maximum(acc, h), acc)
    h = jnp.dot(w[:, :2 * CIN], x[CIN * (WLEN - 2):XW, :],
                preferred_element_type=jnp.float32)
    acc = jnp.where(row < COUT, jnp.maximum(acc, h), acc)
    return jnp.tanh(acc + bb)


def _conv_body(x_ref, emb_ref, w_ref, b_ref, char_ref, fwo_ref):
    nl, _, nb = x_ref.shape
    w = w_ref[...]
    bb = b_ref[...]
    row = lax.broadcasted_iota(jnp.int32, (NOUT, nb), 0)
    for k in range(nl):
        acc = _conv_slab(x_ref[k], w, bb, row)
        char_ref[k] = acc
        fwo_ref[k] = jnp.concatenate([acc, jnp.transpose(emb_ref[k])], axis=0)


def _conv_call(x_t, emb3, wbig_t, bcol, nl=1):
    l, _, nb = x_t.shape
    return pl.pallas_call(
        _conv_body,
        grid=(l // nl,),
        in_specs=[
            pl.BlockSpec((nl, XW, nb), lambda i: (i, 0, 0)),
            pl.BlockSpec((nl, nb, D), lambda i: (i, 0, 0)),
            pl.BlockSpec((NOUT, 4 * CIN), lambda i: (0, 0)),
            pl.BlockSpec((NOUT, 1), lambda i: (0, 0)),
        ],
        out_specs=[
            pl.BlockSpec((nl, NOUT, nb), lambda i: (i, 0, 0)),
            pl.BlockSpec((nl, NOUT + D, nb), lambda i: (i, 0, 0)),
        ],
        out_shape=[
            jax.ShapeDtypeStruct((l, NOUT, nb), jnp.float32),
            jax.ShapeDtypeStruct((l, NOUT + D, nb), jnp.float32),
        ],
    )(x_t, emb3, wbig_t, bcol)


@functools.cache
def _make_gather(vocab, d, t):
    info = plsc.get_sparse_core_info()
    nw = info.num_cores * info.num_subcores
    t_per_w = t // nw
    ch = 400
    n_ch = t_per_w // ch
    mesh = plsc.VectorSubcoreMesh(core_axis_name="c", subcore_axis_name="s")

    @functools.partial(
        pl.kernel,
        mesh=mesh,
        out_type=jax.ShapeDtypeStruct((t, d), jnp.float32),
        scratch_types=[
            pltpu.VMEM((t_per_w,), jnp.int32),
            pltpu.VMEM((2, ch, d), jnp.float32),
            pltpu.SemaphoreType.DMA,
            pltpu.SemaphoreType.DMA,
        ],
    )
    def gather_k(idx_hbm, table_hbm, out_hbm, idx_v, rows_v, sem0, sem1):
        wid = lax.axis_index("s") * info.num_cores + lax.axis_index("c")
        base = wid * t_per_w
        pltpu.sync_copy(idx_hbm.at[pl.ds(base, t_per_w)], idx_v)
        sems = (sem0, sem1)
        cps = [None, None]
        cps[0] = pltpu.async_copy(
            table_hbm.at[idx_v.at[pl.ds(0, ch)]], rows_v.at[0], sems[0]
        )
        for c in range(n_ch):
            nxt = c + 1
            if nxt < n_ch:
                cps[nxt % 2] = pltpu.async_copy(
                    table_hbm.at[idx_v.at[pl.ds(nxt * ch, ch)]],
                    rows_v.at[nxt % 2],
                    sems[nxt % 2],
                )
            cps[c % 2].wait()
            pltpu.sync_copy(rows_v.at[c % 2], out_hbm.at[pl.ds(base + c * ch, ch)])

    return gather_k


def kernel(x0_word_ids, x1_char_feats, table, W2, b2, W3, b3, W4, b4):
    b, l = x0_word_ids.shape
    t = b * l
    idx = jnp.transpose(x0_word_ids).reshape(t).astype(jnp.int32)
    emb_flat = _make_gather(table.shape[0], D, t)(idx, table)
    emb3 = emb_flat.reshape(l, b, D)

    w2p = jnp.pad(W2, ((0, 2 * CIN), (0, 0)))
    w3p = jnp.pad(W3, ((0, CIN), (0, 0)))
    wbig_t = jnp.transpose(jnp.concatenate([w2p, w3p, W4], axis=1))
    bcol = jnp.concatenate([b2, b3, b4]).reshape(NOUT, 1)

    x_t = jnp.transpose(x1_char_feats, (1, 2, 3, 0)).reshape(l, XW, b)
    char_t, fwo_t = _conv_call(x_t, emb3, wbig_t, bcol, nl=5)

    return (
        jnp.transpose(emb3, (1, 0, 2)),
        jnp.transpose(char_t, (2, 0, 1)),
        jnp.transpose(fwo_t, (2, 0, 1)),
    )


# speedup vs baseline: 1.1146x; 1.0047x over previous
import functools

import jax
import jax.numpy as jnp
from jax import lax
from jax.experimental import pallas as pl
from jax.experimental.pallas import tpu as pltpu
from jax.experimental.pallas import tpu_sc as plsc

D = 128
CIN = 32
COUT = 64
WLEN = 16
NOUT = 3 * COUT
XW = WLEN * CIN


def _conv_slab(x, w, bb, row):
    acc = None
    for p in range(WLEN - 3):
        h = jnp.dot(w, x[CIN * p:CIN * p + 4 * CIN, :],
                    preferred_element_type=jnp.float32)
        acc = h if acc is None else jnp.maximum(acc, h)
    h = jnp.dot(w[:, :3 * CIN], x[CIN * (WLEN - 3):XW, :],
                preferred_element_type=jnp.float32)
    acc = jnp.where(row < 2 * COUT, jnp.maximum(acc, h), acc)
    h = jnp.dot(w[:, :2 * CIN], x[CIN * (WLEN - 2):XW, :],
                preferred_element_type=jnp.float32)
    acc = jnp.where(row < COUT, jnp.maximum(acc, h), acc)
    return jnp.tanh(acc + bb)


def _conv_body(x_ref, emb_ref, w_ref, b_ref, char_ref, fwo_ref):
    nl, _, nb = x_ref.shape
    w = w_ref[...]
    bb = b_ref[...]
    row = lax.broadcasted_iota(jnp.int32, (NOUT, nb), 0)
    for k in range(nl):
        acc = _conv_slab(x_ref[k], w, bb, row)
        char_ref[k] = acc
        fwo_ref[k] = jnp.concatenate([acc, jnp.transpose(emb_ref[k])], axis=0)


def _conv_call(x_t, emb3, wbig_t, bcol, nl=1):
    l, _, nb = x_t.shape
    return pl.pallas_call(
        _conv_body,
        grid=(l // nl,),
        in_specs=[
            pl.BlockSpec((nl, XW, nb), lambda i: (i, 0, 0)),
            pl.BlockSpec((nl, nb, D), lambda i: (i, 0, 0)),
            pl.BlockSpec((NOUT, 4 * CIN), lambda i: (0, 0)),
            pl.BlockSpec((NOUT, 1), lambda i: (0, 0)),
        ],
        out_specs=[
            pl.BlockSpec((nl, NOUT, nb), lambda i: (i, 0, 0)),
            pl.BlockSpec((nl, NOUT + D, nb), lambda i: (i, 0, 0)),
        ],
        out_shape=[
            jax.ShapeDtypeStruct((l, NOUT, nb), jnp.float32),
            jax.ShapeDtypeStruct((l, NOUT + D, nb), jnp.float32),
        ],
    )(x_t, emb3, wbig_t, bcol)


@functools.cache
def _make_gather(vocab, d, t):
    info = plsc.get_sparse_core_info()
    nw = info.num_cores * info.num_subcores
    t_per_w = t // nw
    ch = 200
    n_ch = t_per_w // ch
    mesh = plsc.VectorSubcoreMesh(core_axis_name="c", subcore_axis_name="s")

    @functools.partial(
        pl.kernel,
        mesh=mesh,
        out_type=jax.ShapeDtypeStruct((t, d), jnp.float32),
        scratch_types=[
            pltpu.VMEM((t_per_w,), jnp.int32),
            pltpu.VMEM((2, ch, d), jnp.float32),
            pltpu.SemaphoreType.DMA,
            pltpu.SemaphoreType.DMA,
        ],
    )
    def gather_k(idx_hbm, table_hbm, out_hbm, idx_v, rows_v, sem0, sem1):
        wid = lax.axis_index("s") * info.num_cores + lax.axis_index("c")
        base = wid * t_per_w
        pltpu.sync_copy(idx_hbm.at[pl.ds(base, t_per_w)], idx_v)
        sems = (sem0, sem1)
        cps = [None, None]
        cps[0] = pltpu.async_copy(
            table_hbm.at[idx_v.at[pl.ds(0, ch)]], rows_v.at[0], sems[0]
        )
        for c in range(n_ch):
            nxt = c + 1
            if nxt < n_ch:
                cps[nxt % 2] = pltpu.async_copy(
                    table_hbm.at[idx_v.at[pl.ds(nxt * ch, ch)]],
                    rows_v.at[nxt % 2],
                    sems[nxt % 2],
                )
            cps[c % 2].wait()
            pltpu.sync_copy(rows_v.at[c % 2], out_hbm.at[pl.ds(base + c * ch, ch)])

    return gather_k


def kernel(x0_word_ids, x1_char_feats, table, W2, b2, W3, b3, W4, b4):
    b, l = x0_word_ids.shape
    t = b * l
    idx = jnp.transpose(x0_word_ids).reshape(t).astype(jnp.int32)
    emb_flat = _make_gather(table.shape[0], D, t)(idx, table)
    emb3 = emb_flat.reshape(l, b, D)

    w2p = jnp.pad(W2, ((0, 2 * CIN), (0, 0)))
    w3p = jnp.pad(W3, ((0, CIN), (0, 0)))
    wbig_t = jnp.transpose(jnp.concatenate([w2p, w3p, W4], axis=1))
    bcol = jnp.concatenate([b2, b3, b4]).reshape(NOUT, 1)

    x_t = jnp.transpose(x1_char_feats, (1, 2, 3, 0)).reshape(l, XW, b)
    char_t, fwo_t = _conv_call(x_t, emb3, wbig_t, bcol, nl=5)

    return (
        jnp.transpose(emb3, (1, 0, 2)),
        jnp.transpose(char_t, (2, 0, 1)),
        jnp.transpose(fwo_t, (2, 0, 1)),
    )
